# baseline (speedup 1.0000x reference)
import functools

import jax
import jax.numpy as jnp
from jax import lax
from jax.experimental import pallas as pl
from jax.experimental.pallas import tpu as pltpu
from jax.experimental.pallas import tpu_sc as plsc


def _build_gather(n_rows: int, d: int, n_table_rows: int):
    info = plsc.get_sparse_core_info()
    nc, ns = info.num_cores, info.num_subcores
    nw = nc * ns
    assert n_rows % nw == 0
    per_w = n_rows // nw

    mesh = plsc.VectorSubcoreMesh(core_axis_name="c", subcore_axis_name="s")

    @functools.partial(
        pl.kernel,
        mesh=mesh,
        out_type=jax.ShapeDtypeStruct((n_rows, d), jnp.float32),
        scratch_types=[
            pltpu.VMEM((per_w,), jnp.int32),
            pltpu.VMEM_SHARED((n_table_rows, d), jnp.float32),
            pltpu.SemaphoreType.DMA,
        ],
    )
    def gather_kernel(idx_hbm, table_hbm, out_hbm, idx_v, table_sp, osem):
        wid = lax.axis_index("s") * nc + lax.axis_index("c")
        base = wid * per_w

        @pl.when(lax.axis_index("s") == 0)
        def _():
            pltpu.sync_copy(table_hbm, table_sp)

        pltpu.sync_copy(idx_hbm.at[pl.ds(base, per_w)], idx_v)
        plsc.subcore_barrier()

        def body(g, carry):
            vec = idx_v[pl.ds(g * 16, 16)]
            for l in range(16):
                pltpu.async_copy(table_sp.at[vec[l]],
                                 out_hbm.at[base + g * 16 + l], osem)
            return carry

        lax.fori_loop(0, per_w // 16, body, 0)

        def drain(c, carry):
            pltpu.make_async_copy(table_sp.at[0], out_hbm.at[base], osem).wait()
            return carry

        lax.fori_loop(0, per_w, drain, 0)

    return gather_kernel


def kernel(indices, bank_embedding_weight):
    b, s = indices.shape
    v, d = bank_embedding_weight.shape
    n = b * s
    flat = indices.reshape(n).astype(jnp.int32)
    out = _build_gather(n, d, n_table_rows=v)(flat, bank_embedding_weight)
    return out.reshape(b, s, d)
